# Initial kernel scaffold; baseline (speedup 1.0000x reference)
import jax
import jax.numpy as jnp
from jax.experimental import pallas as pl


def kernel(fea, W_cls, target_points_num):
    raise NotImplementedError("write your pallas kernel here")


# speedup vs baseline: 1.4114x; 1.4114x over previous
import jax
import jax.numpy as jnp
from jax import lax
from jax.experimental import pallas as pl
from jax.experimental.pallas import tpu as pltpu

N = 1048576
D = 64
ROWS2 = N // 2
BLK = 8192
SEL_ROWS = N // 128


def _matvec_kernel(f2_ref, w2_ref, out_ref):
    out_ref[...] = jnp.dot(f2_ref[...], w2_ref[...],
                           preferred_element_type=jnp.float32)


def _matvec(f2, w2):
    grid = (ROWS2 // BLK,)
    return pl.pallas_call(
        _matvec_kernel,
        grid=grid,
        in_specs=[
            pl.BlockSpec((BLK, 128), lambda i: (i, 0)),
            pl.BlockSpec((128, 2), lambda i: (0, 0)),
        ],
        out_specs=pl.BlockSpec((BLK, 2), lambda i: (i, 0)),
        out_shape=jax.ShapeDtypeStruct((ROWS2, 2), jnp.float32),
    )(f2, w2)


def _select_kernel(k_ref, pred_ref, pruned_ref, keep_ref):
    k = k_ref[0]
    pred = pred_ref[...]
    u = lax.bitcast_convert_type(pred, jnp.int32)
    key = jnp.where(u < 0, u ^ jnp.int32(0x7FFFFFFF), u)
    key = key ^ jnp.int32(-2147483648)

    c = jnp.sum((lax.shift_right_logical(key, 31) == 0).astype(jnp.int32))
    take1 = k > c
    rank = jnp.where(take1, k - c, k)
    prefix = jnp.where(take1, jnp.int32(1), jnp.int32(0))
    for b in range(30, -1, -1):
        match = lax.shift_right_logical(key, b + 1) == prefix
        bit0 = (lax.shift_right_logical(key, b) & 1) == 0
        c = jnp.sum((match & bit0).astype(jnp.int32))
        take1 = rank > c
        rank = jnp.where(take1, rank - c, rank)
        prefix = lax.shift_left(prefix, 1) | jnp.where(take1, jnp.int32(1),
                                                       jnp.int32(0))

    up = prefix ^ jnp.int32(-2147483648)
    up = jnp.where(up < 0, up ^ jnp.int32(0x7FFFFFFF), up)
    thres = lax.bitcast_convert_type(up, jnp.float32)

    keep = pred > thres
    pruned_ref[...] = pred * keep.astype(jnp.float32)
    keep_ref[...] = keep.astype(jnp.int8)


def _select(pred2d, k_arr):
    return pl.pallas_call(
        _select_kernel,
        in_specs=[
            pl.BlockSpec(memory_space=pltpu.SMEM),
            pl.BlockSpec(memory_space=pltpu.VMEM),
        ],
        out_shape=[
            jax.ShapeDtypeStruct((SEL_ROWS, 128), jnp.float32),
            jax.ShapeDtypeStruct((SEL_ROWS, 128), jnp.int8),
        ],
    )(k_arr, pred2d)


def kernel(fea, W_cls, target_points_num):
    k_arr = jnp.asarray(N - target_points_num, jnp.int32).reshape(1)
    f2 = fea.reshape(ROWS2, 128)
    w = W_cls[:, 0]
    zeros = jnp.zeros((D,), jnp.float32)
    w2 = jnp.stack([jnp.concatenate([w, zeros]),
                    jnp.concatenate([zeros, w])], axis=1)
    pred2 = _matvec(f2, w2)
    pred2d = pred2.reshape(SEL_ROWS, 128)
    pruned2d, keep2d = _select(pred2d, k_arr)
    pruned = pruned2d.reshape(N, 1)
    keep = keep2d.reshape(N).astype(jnp.bool_)
    return pruned, keep


# speedup vs baseline: 1.6025x; 1.1354x over previous
import jax
import jax.numpy as jnp
from jax import lax
from jax.experimental import pallas as pl
from jax.experimental.pallas import tpu as pltpu

N = 1048576
D = 64
BLK = 8192
SEL_ROWS = N // 128


def _matvec_kernel(fea_ref, w_ref, out_ref):
    out_ref[...] = jnp.dot(fea_ref[...], w_ref[...],
                           preferred_element_type=jnp.float32)


def _matvec(fea, w):
    grid = (N // BLK,)
    return pl.pallas_call(
        _matvec_kernel,
        grid=grid,
        in_specs=[
            pl.BlockSpec((BLK, D), lambda i: (i, 0)),
            pl.BlockSpec((D, 1), lambda i: (0, 0)),
        ],
        out_specs=pl.BlockSpec((BLK, 1), lambda i: (i, 0)),
        out_shape=jax.ShapeDtypeStruct((N, 1), jnp.float32),
    )(fea, w)


def _select_kernel(k_ref, pred_ref, pruned_ref, keep_ref):
    k = k_ref[0]
    pred = pred_ref[...]
    u = lax.bitcast_convert_type(pred, jnp.int32)
    key = jnp.where(u < 0, u ^ jnp.int32(0x7FFFFFFF), u)
    key = key ^ jnp.int32(-2147483648)

    c = jnp.sum((lax.shift_right_logical(key, 31) == 0).astype(jnp.int32))
    take1 = k > c
    rank = jnp.where(take1, k - c, k)
    prefix = jnp.where(take1, jnp.int32(1), jnp.int32(0))
    for b in range(30, -1, -1):
        match = lax.shift_right_logical(key, b + 1) == prefix
        bit0 = (lax.shift_right_logical(key, b) & 1) == 0
        c = jnp.sum((match & bit0).astype(jnp.int32))
        take1 = rank > c
        rank = jnp.where(take1, rank - c, rank)
        prefix = lax.shift_left(prefix, 1) | jnp.where(take1, jnp.int32(1),
                                                       jnp.int32(0))

    up = prefix ^ jnp.int32(-2147483648)
    up = jnp.where(up < 0, up ^ jnp.int32(0x7FFFFFFF), up)
    thres = lax.bitcast_convert_type(up, jnp.float32)

    keep = pred > thres
    pruned_ref[...] = pred * keep.astype(jnp.float32)
    keep_ref[...] = keep.astype(jnp.int8)


def _select(pred2d, k_arr):
    return pl.pallas_call(
        _select_kernel,
        in_specs=[
            pl.BlockSpec(memory_space=pltpu.SMEM),
            pl.BlockSpec(memory_space=pltpu.VMEM),
        ],
        out_shape=[
            jax.ShapeDtypeStruct((SEL_ROWS, 128), jnp.float32),
            jax.ShapeDtypeStruct((SEL_ROWS, 128), jnp.int8),
        ],
    )(k_arr, pred2d)


def kernel(fea, W_cls, target_points_num):
    k_arr = jnp.asarray(N - target_points_num, jnp.int32).reshape(1)
    pred = _matvec(fea, W_cls)
    pred2d = pred.reshape(SEL_ROWS, 128)
    pruned2d, keep2d = _select(pred2d, k_arr)
    pruned = pruned2d.reshape(N, 1)
    keep = keep2d.reshape(N).astype(jnp.bool_)
    return pruned, keep


# speedup vs baseline: 1.6049x; 1.0015x over previous
import jax
import jax.numpy as jnp
from jax import lax
from jax.experimental import pallas as pl
from jax.experimental.pallas import tpu as pltpu

N = 1048576
D = 64
BLK = 16384
SEL_ROWS = N // 128


def _matvec_kernel(fea_ref, w_ref, out_ref):
    out_ref[...] = jnp.dot(fea_ref[...], w_ref[...],
                           preferred_element_type=jnp.float32)


def _matvec(fea, w):
    grid = (N // BLK,)
    return pl.pallas_call(
        _matvec_kernel,
        grid=grid,
        in_specs=[
            pl.BlockSpec((BLK, D), lambda i: (i, 0)),
            pl.BlockSpec((D, 1), lambda i: (0, 0)),
        ],
        out_specs=pl.BlockSpec((BLK, 1), lambda i: (i, 0)),
        out_shape=jax.ShapeDtypeStruct((N, 1), jnp.float32),
    )(fea, w)


def _select_kernel(k_ref, pred_ref, pruned_ref, keep_ref):
    k = k_ref[0]
    pred = pred_ref[...]
    u = lax.bitcast_convert_type(pred, jnp.int32)
    key = jnp.where(u < 0, u ^ jnp.int32(0x7FFFFFFF), u)
    key = key ^ jnp.int32(-2147483648)

    c = jnp.sum((lax.shift_right_logical(key, 31) == 0).astype(jnp.int32))
    take1 = k > c
    rank = jnp.where(take1, k - c, k)
    prefix = jnp.where(take1, jnp.int32(1), jnp.int32(0))
    for b in range(30, -1, -1):
        match = lax.shift_right_logical(key, b + 1) == prefix
        bit0 = (lax.shift_right_logical(key, b) & 1) == 0
        c = jnp.sum((match & bit0).astype(jnp.int32))
        take1 = rank > c
        rank = jnp.where(take1, rank - c, rank)
        prefix = lax.shift_left(prefix, 1) | jnp.where(take1, jnp.int32(1),
                                                       jnp.int32(0))

    up = prefix ^ jnp.int32(-2147483648)
    up = jnp.where(up < 0, up ^ jnp.int32(0x7FFFFFFF), up)
    thres = lax.bitcast_convert_type(up, jnp.float32)

    keep = pred > thres
    pruned_ref[...] = pred * keep.astype(jnp.float32)
    keep_ref[...] = keep.astype(jnp.int8)


def _select(pred2d, k_arr):
    return pl.pallas_call(
        _select_kernel,
        in_specs=[
            pl.BlockSpec(memory_space=pltpu.SMEM),
            pl.BlockSpec(memory_space=pltpu.VMEM),
        ],
        out_shape=[
            jax.ShapeDtypeStruct((SEL_ROWS, 128), jnp.float32),
            jax.ShapeDtypeStruct((SEL_ROWS, 128), jnp.int8),
        ],
    )(k_arr, pred2d)


def kernel(fea, W_cls, target_points_num):
    k_arr = jnp.asarray(N - target_points_num, jnp.int32).reshape(1)
    pred = _matvec(fea, W_cls)
    pred2d = pred.reshape(SEL_ROWS, 128)
    pruned2d, keep2d = _select(pred2d, k_arr)
    pruned = pruned2d.reshape(N, 1)
    keep = keep2d.reshape(N).astype(jnp.bool_)
    return pruned, keep


# speedup vs baseline: 2.4386x; 1.5195x over previous
import jax
import jax.numpy as jnp
from jax import lax
from jax.experimental import pallas as pl
from jax.experimental.pallas import tpu as pltpu

N = 1048576
D = 64
BLK = 16384
SEL_ROWS = N // 128


def _matvec_kernel(fea_ref, w_ref, out_ref):
    s = jnp.dot(fea_ref[...], w_ref[...],
                preferred_element_type=jnp.float32)
    out_ref[...] = s.reshape(BLK // 128, 128)


def _matvec(fea, w):
    grid = (N // BLK,)
    return pl.pallas_call(
        _matvec_kernel,
        grid=grid,
        in_specs=[
            pl.BlockSpec((BLK, D), lambda i: (i, 0)),
            pl.BlockSpec((D, 1), lambda i: (0, 0)),
        ],
        out_specs=pl.BlockSpec((BLK // 128, 128), lambda i: (i, 0)),
        out_shape=jax.ShapeDtypeStruct((SEL_ROWS, 128), jnp.float32),
    )(fea, w)


def _select_kernel(k_ref, pred_ref, pruned_ref, keep_ref):
    k = k_ref[0]
    pred = pred_ref[...]
    u = lax.bitcast_convert_type(pred, jnp.int32)
    key = jnp.where(u < 0, u ^ jnp.int32(0x7FFFFFFF), u)
    key = key ^ jnp.int32(-2147483648)

    c = jnp.sum((lax.shift_right_logical(key, 31) == 0).astype(jnp.int32))
    take1 = k > c
    rank = jnp.where(take1, k - c, k)
    prefix = jnp.where(take1, jnp.int32(1), jnp.int32(0))
    for b in range(30, -1, -1):
        match = lax.shift_right_logical(key, b + 1) == prefix
        bit0 = (lax.shift_right_logical(key, b) & 1) == 0
        c = jnp.sum((match & bit0).astype(jnp.int32))
        take1 = rank > c
        rank = jnp.where(take1, rank - c, rank)
        prefix = lax.shift_left(prefix, 1) | jnp.where(take1, jnp.int32(1),
                                                       jnp.int32(0))

    up = prefix ^ jnp.int32(-2147483648)
    up = jnp.where(up < 0, up ^ jnp.int32(0x7FFFFFFF), up)
    thres = lax.bitcast_convert_type(up, jnp.float32)

    keep = pred > thres
    pruned_ref[...] = pred * keep.astype(jnp.float32)
    keep_ref[...] = keep.astype(jnp.int8)


def _select(pred2d, k_arr):
    return pl.pallas_call(
        _select_kernel,
        in_specs=[
            pl.BlockSpec(memory_space=pltpu.SMEM),
            pl.BlockSpec(memory_space=pltpu.VMEM),
        ],
        out_shape=[
            jax.ShapeDtypeStruct((SEL_ROWS, 128), jnp.float32),
            jax.ShapeDtypeStruct((SEL_ROWS, 128), jnp.int8),
        ],
    )(k_arr, pred2d)


def kernel(fea, W_cls, target_points_num):
    k_arr = jnp.asarray(N - target_points_num, jnp.int32).reshape(1)
    pred2d = _matvec(fea, W_cls)
    pruned2d, keep2d = _select(pred2d, k_arr)
    pruned = pruned2d.reshape(N, 1)
    keep = keep2d.reshape(N).astype(jnp.bool_)
    return pruned, keep


# speedup vs baseline: 2.4835x; 1.0184x over previous
import jax
import jax.numpy as jnp
from jax import lax
from jax.experimental import pallas as pl
from jax.experimental.pallas import tpu as pltpu

N = 1048576
D = 64
WAYS = 4
BLK = 8192
BLKR = BLK // 128
CH = N // WAYS
STEPS = CH // BLK
SEL_ROWS = N // 128


def _matvec_kernel(f0, f1, f2, f3, w_ref, out_ref):
    w = w_ref[...]
    for c, f in enumerate((f0, f1, f2, f3)):
        s = jnp.dot(f[...], w, preferred_element_type=jnp.float32)
        out_ref[c, :, :] = s.reshape(BLKR, 128)


def _matvec(fea, w):
    def fea_spec(c):
        return pl.BlockSpec((BLK, D), lambda i, c=c: (c * STEPS + i, 0))

    return pl.pallas_call(
        _matvec_kernel,
        grid=(STEPS,),
        in_specs=[fea_spec(0), fea_spec(1), fea_spec(2), fea_spec(3),
                  pl.BlockSpec((D, 1), lambda i: (0, 0))],
        out_specs=pl.BlockSpec((WAYS, BLKR, 128), lambda i: (0, i, 0)),
        out_shape=jax.ShapeDtypeStruct((WAYS, SEL_ROWS // WAYS, 128),
                                       jnp.float32),
    )(fea, fea, fea, fea, w)


def _select_kernel(k_ref, pred_ref, pruned_ref, keep_ref):
    k = k_ref[0]
    pred = pred_ref[...]
    u = lax.bitcast_convert_type(pred, jnp.int32)
    key = jnp.where(u < 0, u ^ jnp.int32(0x7FFFFFFF), u)
    key = key ^ jnp.int32(-2147483648)

    c = jnp.sum((lax.shift_right_logical(key, 31) == 0).astype(jnp.int32))
    take1 = k > c
    rank = jnp.where(take1, k - c, k)
    prefix = jnp.where(take1, jnp.int32(1), jnp.int32(0))
    for b in range(30, -1, -1):
        match = lax.shift_right_logical(key, b + 1) == prefix
        bit0 = (lax.shift_right_logical(key, b) & 1) == 0
        c = jnp.sum((match & bit0).astype(jnp.int32))
        take1 = rank > c
        rank = jnp.where(take1, rank - c, rank)
        prefix = lax.shift_left(prefix, 1) | jnp.where(take1, jnp.int32(1),
                                                       jnp.int32(0))

    up = prefix ^ jnp.int32(-2147483648)
    up = jnp.where(up < 0, up ^ jnp.int32(0x7FFFFFFF), up)
    thres = lax.bitcast_convert_type(up, jnp.float32)

    keep = pred > thres
    pruned_ref[...] = pred * keep.astype(jnp.float32)
    keep_ref[...] = keep.astype(jnp.int8)


def _select(pred3, k_arr):
    return pl.pallas_call(
        _select_kernel,
        in_specs=[
            pl.BlockSpec(memory_space=pltpu.SMEM),
            pl.BlockSpec(memory_space=pltpu.VMEM),
        ],
        out_shape=[
            jax.ShapeDtypeStruct(pred3.shape, jnp.float32),
            jax.ShapeDtypeStruct(pred3.shape, jnp.int8),
        ],
    )(k_arr, pred3)


def kernel(fea, W_cls, target_points_num):
    k_arr = jnp.asarray(N - target_points_num, jnp.int32).reshape(1)
    pred3 = _matvec(fea, W_cls)
    pruned3, keep3 = _select(pred3, k_arr)
    pruned = pruned3.reshape(N, 1)
    keep = keep3.reshape(N).astype(jnp.bool_)
    return pruned, keep
